# baseline (speedup 1.0000x reference)
import functools

import jax
import jax.numpy as jnp
from jax import lax
from jax.experimental import pallas as pl
from jax.experimental.pallas import tpu as pltpu
from jax.experimental.pallas import tpu_sc as plsc

T = 2048
D = 2048
DFF = 4096
E = 8
TILE = 256
NT = 24
LP = NT * TILE
VP = LP - 2 * T
FBLK = 1024
NF = DFF // FBLK
DBLK = 1024
ND = D // DBLK
NEG = -1e30


def _router_body(x_ref, wgp_ref, wg_ref, bg_ref, packed_ref, meta_ref,
                 hd_ref, loss_ref):
    x = x_ref[...]
    wgp = wgp_ref[...]
    logits = lax.dot_general(x, wgp, (((1,), (1,)), ((), ())),
                             preferred_element_type=jnp.float32)
    logits = logits + bg_ref[...]
    lane = lax.broadcasted_iota(jnp.int32, (T, 128), 1)

    m1 = jnp.max(logits, axis=1, keepdims=True)
    i1 = jnp.min(jnp.where(logits == m1, lane, 128), axis=1, keepdims=True)
    sel1 = lane == i1
    logits2 = jnp.where(sel1, NEG, logits)
    m2 = jnp.max(logits2, axis=1, keepdims=True)
    i2 = jnp.min(jnp.where(logits2 == m2, lane, 128), axis=1, keepdims=True)
    sel2 = lane == i2
    ed = jnp.exp(m2 - m1)
    w1v = 1.0 / (1.0 + ed)
    w2v = ed / (1.0 + ed)

    ind0 = sel1.astype(jnp.float32)
    ind1 = sel2.astype(jnp.float32)
    ti = lax.broadcasted_iota(jnp.int32, (T, T), 0)
    tj = lax.broadcasted_iota(jnp.int32, (T, T), 1)
    ltri = (ti > tj).astype(jnp.float32)
    pre0 = lax.dot_general(ltri, ind0, (((1,), (0,)), ((), ())),
                           preferred_element_type=jnp.float32)
    pre1 = lax.dot_general(ltri, ind1, (((1,), (0,)), ((), ())),
                           preferred_element_type=jnp.float32)
    cnt0 = jnp.sum(ind0, axis=0, keepdims=True)
    cnt1 = jnp.sum(ind1, axis=0, keepdims=True)
    counts = cnt0 + cnt1
    pc = jnp.ceil(counts / TILE) * TILE
    li = lax.broadcasted_iota(jnp.int32, (128, 128), 0)
    lj = lax.broadcasted_iota(jnp.int32, (128, 128), 1)
    ltl = (li < lj).astype(jnp.float32)
    off = lax.dot_general(pc, ltl, (((1,), (0,)), ((), ())),
                          preferred_element_type=jnp.float32)
    cum = off + pc
    hc = pc - counts
    hb = lax.dot_general(hc, ltl, (((1,), (0,)), ((), ())),
                         preferred_element_type=jnp.float32)

    rank0 = jnp.sum(ind0 * pre0, axis=1, keepdims=True)
    rank1 = jnp.sum(ind1 * (pre1 + cnt0), axis=1, keepdims=True)
    off0 = jnp.sum(ind0 * off, axis=1, keepdims=True)
    off1 = jnp.sum(ind1 * off, axis=1, keepdims=True)
    d0 = (off0 + rank0).astype(jnp.int32)
    d1 = (off1 + rank1).astype(jnp.int32)

    lane_row = lane[:1]
    jstart = (lane_row * TILE).astype(jnp.float32)
    te = jnp.zeros((1, 128), jnp.float32)
    hd = jnp.zeros((16, 128), jnp.float32)
    vr = lax.broadcasted_iota(jnp.int32, (16, 128), 0)
    vl = lax.broadcasted_iota(jnp.int32, (16, 128), 1)
    vlin = (vr * 128 + vl).astype(jnp.float32)
    hb8 = jnp.sum(hc)
    cum7 = jnp.sum(pc)
    loss = jnp.float32(0.0)

    wg = wg_ref[...]
    mx = jnp.max(wg, axis=0, keepdims=True)
    ex = jnp.exp(wg - mx)
    pr = ex / jnp.sum(ex, axis=0, keepdims=True)
    rp = jnp.mean(pr, axis=1, keepdims=True)
    sub = lax.broadcasted_iota(jnp.int32, (E, 1), 0)

    for e in range(E):
        sel_e = (lane_row == e).astype(jnp.float32)
        cum_e = jnp.sum(cum * sel_e)
        te = te + (jstart >= cum_e).astype(jnp.float32)
        hb_e = jnp.sum(hb * sel_e)
        hc_e = jnp.sum(hc * sel_e)
        base_e = jnp.sum((off + counts) * sel_e)
        in_e = jnp.logical_and(vlin >= hb_e, vlin < hb_e + hc_e)
        hd = hd + jnp.where(in_e, base_e + vlin - hb_e, 0.0)
        c_e = jnp.sum(counts * sel_e)
        rp_e = jnp.sum(rp * (sub == e).astype(jnp.float32))
        loss = loss + c_e * rp_e
    hd = hd + jnp.where(vlin >= hb8, cum7 + vlin - hb8, 0.0)
    te = jnp.minimum(te, 7.0)

    w1b = lax.bitcast_convert_type(w1v, jnp.int32)
    w2b = lax.bitcast_convert_type(w2v, jnp.int32)
    packed = jnp.where(lane == 0, d0, 0)
    packed = packed + jnp.where(lane == 1, d1, 0)
    packed = packed + jnp.where(lane == 2, w1b, 0)
    packed = packed + jnp.where(lane == 3, w2b, 0)
    packed_ref[...] = packed
    meta_ref[...] = te.astype(jnp.int32)
    hd_ref[...] = hd.astype(jnp.int32)
    loss_ref[...] = jnp.full((1, 1), loss * (float(E) / T), jnp.float32)


def _router(x2d, wg_pad, wg, bg_row):
    return pl.pallas_call(
        _router_body,
        out_shape=[
            jax.ShapeDtypeStruct((T, 128), jnp.int32),
            jax.ShapeDtypeStruct((1, 128), jnp.int32),
            jax.ShapeDtypeStruct((16, 128), jnp.int32),
            jax.ShapeDtypeStruct((1, 1), jnp.float32),
        ],
    )(x2d, wg_pad, wg, bg_row)


def _gelu(a):
    return 0.5 * a * (1.0 + lax.erf(a * 0.7071067811865476))


def _ffn1_body(te_ref, xg_ref, w1_ref, b1_ref, h_ref):
    x = xg_ref[...]
    w = jnp.squeeze(w1_ref[...], 0)
    b = jnp.squeeze(b1_ref[...], 0)
    acc = lax.dot_general(x, w, (((1,), (1,)), ((), ())),
                          preferred_element_type=jnp.float32)
    h_ref[...] = _gelu(acc + b)


def _ffn1(te, xg, w1, b1r):
    grid_spec = pltpu.PrefetchScalarGridSpec(
        num_scalar_prefetch=1,
        grid=(NF, NT),
        in_specs=[
            pl.BlockSpec((TILE, D), lambda f, t, te: (t, 0)),
            pl.BlockSpec((1, FBLK, D), lambda f, t, te: (te[t], f, 0)),
            pl.BlockSpec((1, 1, FBLK), lambda f, t, te: (te[t] * NF + f, 0, 0)),
        ],
        out_specs=pl.BlockSpec((TILE, FBLK), lambda f, t, te: (t, f)),
    )
    return pl.pallas_call(
        _ffn1_body,
        grid_spec=grid_spec,
        out_shape=jax.ShapeDtypeStruct((LP, DFF), jnp.float32),
    )(te, xg, w1, b1r)


def _ffn2_body(te_ref, h_ref, w2_ref, b2_ref, sw_ref, o_ref):
    h = h_ref[...]
    w = jnp.squeeze(w2_ref[...], 0)
    b = jnp.squeeze(b2_ref[...], 0)
    acc = lax.dot_general(h, w, (((1,), (1,)), ((), ())),
                          preferred_element_type=jnp.float32)
    o_ref[...] = (acc + b) * sw_ref[...]


def _ffn2(te, h, w2, b2r, sw):
    grid_spec = pltpu.PrefetchScalarGridSpec(
        num_scalar_prefetch=1,
        grid=(ND, NT),
        in_specs=[
            pl.BlockSpec((TILE, DFF), lambda d, t, te: (t, 0)),
            pl.BlockSpec((1, DBLK, DFF), lambda d, t, te: (te[t], d, 0)),
            pl.BlockSpec((1, 1, DBLK), lambda d, t, te: (te[t] * ND + d, 0, 0)),
            pl.BlockSpec((TILE, 1), lambda d, t, te: (t, 0)),
        ],
        out_specs=pl.BlockSpec((TILE, DBLK), lambda d, t, te: (t, d)),
    )
    return pl.pallas_call(
        _ffn2_body,
        grid_spec=grid_spec,
        out_shape=jax.ShapeDtypeStruct((LP, D), jnp.float32),
    )(te, h, w2, b2r, sw)


NW = 32


def _dispatch_sc(x2d, srcs, dsts, wall):
    ch = LP // NW
    nch = ch // 16
    mesh = plsc.VectorSubcoreMesh(core_axis_name="c", subcore_axis_name="s")

    @functools.partial(
        pl.kernel, mesh=mesh,
        out_type=[jax.ShapeDtypeStruct((LP, D), jnp.float32),
                  jax.ShapeDtypeStruct((LP,), jnp.float32)],
        scratch_types=[pltpu.VMEM((16,), jnp.int32),
                       pltpu.VMEM((16,), jnp.int32),
                       pltpu.VMEM((16,), jnp.float32),
                       pltpu.VMEM((16, D), jnp.float32),
                       pltpu.SemaphoreType.DMA],
    )
    def k(x_hbm, s_hbm, d_hbm, w_hbm, xg_hbm, sw_hbm, sv, dv, wv, rows, sem):
        wid = lax.axis_index("s") * 2 + lax.axis_index("c")
        base = wid * ch

        def body(i, carry):
            off = base + i * 16
            pltpu.sync_copy(s_hbm.at[pl.ds(off, 16)], sv)
            pltpu.sync_copy(d_hbm.at[pl.ds(off, 16)], dv)
            pltpu.sync_copy(w_hbm.at[pl.ds(off, 16)], wv)
            pltpu.async_copy(x_hbm.at[sv], rows, sem).wait()
            pltpu.async_copy(rows, xg_hbm.at[dv], sem).wait()
            pltpu.async_copy(wv, sw_hbm.at[dv], sem).wait()
            return carry

        lax.fori_loop(0, nch, body, 0)

    return k(x2d, srcs, dsts, wall)


def _combine_sc(outs, d0, d1):
    ch = T // NW
    nch = ch // 16
    mesh = plsc.VectorSubcoreMesh(core_axis_name="c", subcore_axis_name="s")

    @functools.partial(
        pl.kernel, mesh=mesh,
        out_type=[jax.ShapeDtypeStruct((T, D), jnp.float32),
                  jax.ShapeDtypeStruct((T, D), jnp.float32)],
        scratch_types=[pltpu.VMEM((16,), jnp.int32),
                       pltpu.VMEM((16, D), jnp.float32),
                       pltpu.SemaphoreType.DMA],
    )
    def k(o_hbm, d0_hbm, d1_hbm, g0_hbm, g1_hbm, iv, rows, sem):
        wid = lax.axis_index("s") * 2 + lax.axis_index("c")
        base = wid * ch

        def body(i, carry):
            off = base + i * 16
            pltpu.sync_copy(d0_hbm.at[pl.ds(off, 16)], iv)
            pltpu.async_copy(o_hbm.at[iv], rows, sem).wait()
            pltpu.sync_copy(rows, g0_hbm.at[pl.ds(off, 16)])
            pltpu.sync_copy(d1_hbm.at[pl.ds(off, 16)], iv)
            pltpu.async_copy(o_hbm.at[iv], rows, sem).wait()
            pltpu.sync_copy(rows, g1_hbm.at[pl.ds(off, 16)])
            return carry

        lax.fori_loop(0, nch, body, 0)

    return k(outs, d0, d1)


def _add_body(a_ref, b_ref, o_ref):
    o_ref[...] = a_ref[...] + b_ref[...]


def _add(a, b):
    return pl.pallas_call(
        _add_body,
        grid=(8,),
        in_specs=[pl.BlockSpec((T // 8, D), lambda i: (i, 0)),
                  pl.BlockSpec((T // 8, D), lambda i: (i, 0))],
        out_specs=pl.BlockSpec((T // 8, D), lambda i: (i, 0)),
        out_shape=jax.ShapeDtypeStruct((T, D), jnp.float32),
    )(a, b)


def kernel(x, Wg, bg, W1, b1, W2, b2):
    x2d = x.reshape(T, D)
    wg_pad = jnp.zeros((128, D), jnp.float32).at[:E].set(Wg)
    bg_row = jnp.full((1, 128), NEG, jnp.float32).at[0, :E].set(bg)
    packed, meta, hd, loss = _router(x2d, wg_pad, Wg, bg_row)

    d0 = packed[:, 0]
    d1 = packed[:, 1]
    w1v = lax.bitcast_convert_type(packed[:, 2], jnp.float32)
    w2v = lax.bitcast_convert_type(packed[:, 3], jnp.float32)
    te = meta[0, :NT]

    tok = jnp.arange(T, dtype=jnp.int32)
    srcs = jnp.concatenate([tok, tok, jnp.zeros((VP,), jnp.int32)])
    dsts = jnp.concatenate([d0, d1, hd.reshape(-1)])
    wall = jnp.concatenate([w1v, w2v, jnp.zeros((VP,), jnp.float32)])

    xg, sw = _dispatch_sc(x2d, srcs, dsts, wall)

    b1r = b1.reshape(E * NF, 1, FBLK)
    b2r = b2.reshape(E * ND, 1, DBLK)
    h = _ffn1(te, xg, W1, b1r)
    outs = _ffn2(te, h, W2, b2r, sw.reshape(LP, 1))

    g0, g1 = _combine_sc(outs, d0, d1)

    final = _add(g0, g1)
    return final.reshape(x.shape), loss[0, 0]
